# baseline (speedup 1.0000x reference)
import functools

import jax
import jax.numpy as jnp
from jax.experimental import pallas as pl
from jax.experimental.pallas import tpu as pltpu


def _pick_block(n: int) -> int:
    for cand in (400, 200, 80, 40, 16, 8):
        if n % cand == 0:
            return cand
    return n


def _xw_kernel(x_ref, w_ref, o_ref):
    x = x_ref[...].astype(jnp.bfloat16)
    o_ref[...] = jnp.dot(x, w_ref[...], preferred_element_type=jnp.float32).astype(
        jnp.bfloat16
    )


def _layer_kernel(adj_ref, s_ref, b_ref, w2_ref, o_ref, *, fuse_relu_w2: bool):
    a = adj_ref[...].astype(jnp.bfloat16)
    acc = jnp.dot(a, s_ref[...], preferred_element_type=jnp.float32)
    if fuse_relu_w2:
        h = jnp.maximum(acc + b_ref[...], 0.0).astype(jnp.bfloat16)
        o_ref[...] = jnp.dot(h, w2_ref[...], preferred_element_type=jnp.float32).astype(
            jnp.bfloat16
        )
    else:
        o_ref[...] = acc + b_ref[...]


def _layer_call(adj, s, b, w2, *, fuse_relu_w2: bool, out_dtype, bi: int):
    n = adj.shape[0]
    f = s.shape[1]
    fo = w2.shape[1] if fuse_relu_w2 else f
    grid = (n // bi,)
    return pl.pallas_call(
        functools.partial(_layer_kernel, fuse_relu_w2=fuse_relu_w2),
        grid=grid,
        in_specs=[
            pl.BlockSpec((bi, n), lambda i: (i, 0)),
            pl.BlockSpec((n, f), lambda i: (0, 0)),
            pl.BlockSpec((1, f if not fuse_relu_w2 else f), lambda i: (0, 0)),
            pl.BlockSpec((f, fo), lambda i: (0, 0)),
        ],
        out_specs=pl.BlockSpec((bi, fo), lambda i: (i, 0)),
        out_shape=jax.ShapeDtypeStruct((n, fo), out_dtype),
        compiler_params=pltpu.CompilerParams(
            dimension_semantics=("parallel",),
        ),
    )(adj, s, b, w2)


def kernel(x, adj, W1, b1, W2, b2):
    n, nfeat = x.shape
    nhid = W1.shape[1]
    bi = _pick_block(n)

    w1b = W1.astype(jnp.bfloat16)
    w2b = W2.astype(jnp.bfloat16)
    b1r = b1.reshape(1, nhid)
    b2r = b2.reshape(1, W2.shape[1])

    s1 = pl.pallas_call(
        _xw_kernel,
        grid=(1,),
        in_specs=[
            pl.BlockSpec((n, nfeat), lambda i: (0, 0)),
            pl.BlockSpec((nfeat, nhid), lambda i: (0, 0)),
        ],
        out_specs=pl.BlockSpec((n, nhid), lambda i: (0, 0)),
        out_shape=jax.ShapeDtypeStruct((n, nhid), jnp.bfloat16),
    )(x, w1b)

    s2 = _layer_call(
        adj, s1, b1r, w2b, fuse_relu_w2=True, out_dtype=jnp.bfloat16, bi=bi
    )
    out = _layer_call(
        adj, s2, b2r, w2b, fuse_relu_w2=False, out_dtype=jnp.float32, bi=bi
    )
    return out


# speedup vs baseline: 1.0049x; 1.0049x over previous
import functools

import jax
import jax.numpy as jnp
from jax.experimental import pallas as pl
from jax.experimental.pallas import tpu as pltpu


def _pick_block(n: int) -> int:
    for cand in (400, 200, 80, 40, 16, 8):
        if n % cand == 0:
            return cand
    return n


def _xw_kernel(x_ref, w_ref, o_ref):
    o_ref[...] = jnp.dot(x_ref[...], w_ref[...], preferred_element_type=jnp.float32)


def _layer_kernel(adj_ref, s_ref, b_ref, w2_ref, o_ref, *, fuse_relu_w2: bool):
    acc = jnp.dot(adj_ref[...], s_ref[...], preferred_element_type=jnp.float32)
    if fuse_relu_w2:
        h = jnp.maximum(acc + b_ref[...], 0.0)
        o_ref[...] = jnp.dot(h, w2_ref[...], preferred_element_type=jnp.float32)
    else:
        o_ref[...] = acc + b_ref[...]


def _layer_call(adj, s, b, w2, *, fuse_relu_w2: bool, bi: int):
    n = adj.shape[0]
    f = s.shape[1]
    fo = w2.shape[1] if fuse_relu_w2 else f
    grid = (n // bi,)
    return pl.pallas_call(
        functools.partial(_layer_kernel, fuse_relu_w2=fuse_relu_w2),
        grid=grid,
        in_specs=[
            pl.BlockSpec((bi, n), lambda i: (i, 0)),
            pl.BlockSpec((n, f), lambda i: (0, 0)),
            pl.BlockSpec((1, f), lambda i: (0, 0)),
            pl.BlockSpec((f, fo), lambda i: (0, 0)),
        ],
        out_specs=pl.BlockSpec((bi, fo), lambda i: (i, 0)),
        out_shape=jax.ShapeDtypeStruct((n, fo), jnp.float32),
        compiler_params=pltpu.CompilerParams(
            dimension_semantics=("arbitrary",),
        ),
    )(adj, s, b, w2)


def kernel(x, adj, W1, b1, W2, b2):
    n, nfeat = x.shape
    nhid = W1.shape[1]
    bi = _pick_block(n)

    b1r = b1.reshape(1, nhid)
    b2r = b2.reshape(1, W2.shape[1])

    s1 = pl.pallas_call(
        _xw_kernel,
        grid=(1,),
        in_specs=[
            pl.BlockSpec((n, nfeat), lambda i: (0, 0)),
            pl.BlockSpec((nfeat, nhid), lambda i: (0, 0)),
        ],
        out_specs=pl.BlockSpec((n, nhid), lambda i: (0, 0)),
        out_shape=jax.ShapeDtypeStruct((n, nhid), jnp.float32),
    )(x, W1)

    s2 = _layer_call(adj, s1, b1r, W2, fuse_relu_w2=True, bi=bi)
    out = _layer_call(adj, s2, b2r, W2, fuse_relu_w2=False, bi=bi)
    return out
